# Initial kernel scaffold; baseline (speedup 1.0000x reference)
import jax
import jax.numpy as jnp
from jax.experimental import pallas as pl


def kernel(x, edge_index, batch, W1, a1s, a1d, b1, W2, a2s, a2d, b2, Wl, bl):
    raise NotImplementedError("write your pallas kernel here")


# speedup vs baseline: 31.0637x; 31.0637x over previous
import functools

import jax
import jax.numpy as jnp
from jax import lax
from jax.experimental import pallas as pl
from jax.experimental.pallas import tpu as pltpu
from jax.experimental.pallas import tpu_sc as plsc

N = 50000
E = 800000
IN_DIM = 768
H = 64
HH = 32
G = 256
OUT = 2

RB = 2000
NBLK = N // RB

NSC = 2
NT = 16
IDXW = 128
KROW = 5
CH = IDXW * KROW
NCHUNK = E // CH
CPT = (NCHUNK + NT - 1) // NT
ROWS_T = N // NT
DHALF = N // 2

_EPS = 1e-16


def _tc_finish(h, A_ref, i, h0_ref, h1_ref, aux_ref, m_ref):
    aux = jnp.dot(h, A_ref[...], preferred_element_type=jnp.float32)
    h0_ref[...] = h[:, :HH]
    h1_ref[...] = h[:, HH:]
    aux_ref[...] = aux[None]
    mb = jnp.max(aux, axis=0)[None]

    @pl.when(i == 0)
    def _():
        m_ref[...] = mb

    @pl.when(i > 0)
    def _():
        m_ref[...] = jnp.maximum(m_ref[...], mb)


def _k1_body(x_ref, W_ref, A_ref, h0_ref, h1_ref, aux_ref, m_ref):
    i = pl.program_id(0)
    h = jnp.dot(x_ref[...], W_ref[...].T, preferred_element_type=jnp.float32)
    _tc_finish(h, A_ref, i, h0_ref, h1_ref, aux_ref, m_ref)


def _k2_body(accA_ref, accB_ref, den_ref, b_ref, W_ref, A_ref,
             h0_ref, h1_ref, aux_ref, m_ref):
    i = pl.program_id(0)
    inv = 1.0 / (den_ref[0] + _EPS)
    hin = jnp.concatenate([accA_ref[...], accB_ref[...]], axis=1)
    hin = jnp.maximum(hin * inv + b_ref[...], 0.0)
    h = jnp.dot(hin, W_ref[...].T, preferred_element_type=jnp.float32)
    _tc_finish(h, A_ref, i, h0_ref, h1_ref, aux_ref, m_ref)


def _k3_body(accA_ref, accB_ref, den_ref, b_ref, batch_ref, Wl_ref, bl_ref,
             out_ref, sums_ref):
    i = pl.program_id(0)
    inv = 1.0 / (den_ref[0] + _EPS)
    hin = jnp.concatenate([accA_ref[...], accB_ref[...]], axis=1)
    hin = jnp.maximum(hin * inv + b_ref[...], 0.0)
    hin = jnp.concatenate([hin, jnp.ones((RB, 1), jnp.float32)], axis=1)
    bb = batch_ref[0]
    gid = lax.broadcasted_iota(jnp.int32, (1, G), 1)
    onehot = (bb == gid).astype(jnp.float32)
    part = lax.dot_general(onehot, hin, (((0,), (0,)), ((), ())),
                           preferred_element_type=jnp.float32)

    @pl.when(i == 0)
    def _():
        sums_ref[...] = part

    @pl.when(i > 0)
    def _():
        sums_ref[...] = sums_ref[...] + part

    @pl.when(i == NBLK - 1)
    def _():
        s = sums_ref[...]
        g = s[:, :H] / jnp.maximum(s[:, H:], 1.0)
        out_ref[...] = (
            jnp.dot(g, Wl_ref[...].T, preferred_element_type=jnp.float32)
            + bl_ref[...])


_k1 = pl.pallas_call(
    _k1_body,
    grid=(NBLK,),
    in_specs=[
        pl.BlockSpec((RB, IN_DIM), lambda i: (i, 0)),
        pl.BlockSpec((H, IN_DIM), lambda i: (0, 0)),
        pl.BlockSpec((H, 2), lambda i: (0, 0)),
    ],
    out_specs=[
        pl.BlockSpec((RB, HH), lambda i: (i, 0)),
        pl.BlockSpec((RB, HH), lambda i: (i, 0)),
        pl.BlockSpec((1, RB, 2), lambda i: (i, 0, 0)),
        pl.BlockSpec((1, 2), lambda i: (0, 0)),
    ],
    out_shape=[
        jax.ShapeDtypeStruct((N, HH), jnp.float32),
        jax.ShapeDtypeStruct((N, HH), jnp.float32),
        jax.ShapeDtypeStruct((NBLK, RB, 2), jnp.float32),
        jax.ShapeDtypeStruct((1, 2), jnp.float32),
    ],
)

_k2 = pl.pallas_call(
    _k2_body,
    grid=(NBLK,),
    in_specs=[
        pl.BlockSpec((RB, HH), lambda i: (i, 0)),
        pl.BlockSpec((RB, HH), lambda i: (i, 0)),
        pl.BlockSpec((1, RB, 1), lambda i: (i, 0, 0)),
        pl.BlockSpec((1, H), lambda i: (0, 0)),
        pl.BlockSpec((H, H), lambda i: (0, 0)),
        pl.BlockSpec((H, 2), lambda i: (0, 0)),
    ],
    out_specs=[
        pl.BlockSpec((RB, HH), lambda i: (i, 0)),
        pl.BlockSpec((RB, HH), lambda i: (i, 0)),
        pl.BlockSpec((1, RB, 2), lambda i: (i, 0, 0)),
        pl.BlockSpec((1, 2), lambda i: (0, 0)),
    ],
    out_shape=[
        jax.ShapeDtypeStruct((N, HH), jnp.float32),
        jax.ShapeDtypeStruct((N, HH), jnp.float32),
        jax.ShapeDtypeStruct((NBLK, RB, 2), jnp.float32),
        jax.ShapeDtypeStruct((1, 2), jnp.float32),
    ],
)

_k3 = pl.pallas_call(
    _k3_body,
    grid=(NBLK,),
    in_specs=[
        pl.BlockSpec((RB, HH), lambda i: (i, 0)),
        pl.BlockSpec((RB, HH), lambda i: (i, 0)),
        pl.BlockSpec((1, RB, 1), lambda i: (i, 0, 0)),
        pl.BlockSpec((1, H), lambda i: (0, 0)),
        pl.BlockSpec((1, RB, 1), lambda i: (i, 0, 0)),
        pl.BlockSpec((OUT, H), lambda i: (0, 0)),
        pl.BlockSpec((1, OUT), lambda i: (0, 0)),
    ],
    out_specs=pl.BlockSpec((G, OUT), lambda i: (0, 0)),
    out_shape=jax.ShapeDtypeStruct((G, OUT), jnp.float32),
    scratch_shapes=[pltpu.VMEM((G, H + 1), jnp.float32)],
)


def _sc_body(src_hbm, dst_hbm, asv_hbm, adv_hbm, hA_hbm, hB_hbm, m_hbm,
             acc_out, den_out,
             acc_sh, den_sh, srcv, dstv, asg, adg, eev, hrow,
             zden, m_v, sem):
    cid = lax.axis_index("c")
    tid = lax.axis_index("s")

    def _zb(r, c):
        hrow[r, pl.ds(0, 16)] = jnp.zeros((16,), jnp.float32)
        hrow[r, pl.ds(16, 16)] = jnp.zeros((16,), jnp.float32)
        return c
    lax.fori_loop(0, CH, _zb, 0)

    def _zd(r, c):
        zden[pl.ds(r * 16, 16)] = jnp.zeros((16,), jnp.float32)
        return c
    lax.fori_loop(0, 1008 // 16, _zd, 0)

    for k in range(4):
        pltpu.sync_copy(hrow, acc_sh.at[pl.ds(tid * ROWS_T + k * CH, CH)])
    pltpu.sync_copy(hrow.at[pl.ds(0, ROWS_T - 4 * CH)],
                    acc_sh.at[pl.ds(tid * ROWS_T + 4 * CH, ROWS_T - 4 * CH)])

    @pl.when(tid < 2)
    def _():
        def _zdn(k, c):
            pltpu.sync_copy(zden.at[pl.ds(0, 1000)],
                            den_sh.at[pl.ds(tid * DHALF + k * 1000, 1000)])
            return c
        lax.fori_loop(0, DHALF // 1000, _zdn, 0)

    pltpu.sync_copy(m_hbm, m_v)
    plsc.subcore_barrier()

    m_s = m_v[pl.ds(0, 16)][0]

    def _mainloop(htab, do_den):
        base = tid * CPT

        def body(k, c):
            sid = base + k

            @pl.when(sid < NCHUNK)
            def _():
                pltpu.sync_copy(src_hbm.at[sid], srcv)
                pltpu.sync_copy(dst_hbm.at[sid], dstv)
                cps = []
                for j in range(KROW):
                    cps.append(pltpu.async_copy(
                        asv_hbm.at[srcv.at[j]], asg.at[pl.ds(j * IDXW, IDXW)],
                        sem))
                    cps.append(pltpu.async_copy(
                        adv_hbm.at[dstv.at[j]], adg.at[pl.ds(j * IDXW, IDXW)],
                        sem))
                    cps.append(pltpu.async_copy(
                        htab.at[srcv.at[j]], hrow.at[pl.ds(j * IDXW, IDXW)],
                        sem))
                for cp in cps:
                    cp.wait()
                for v in range(CH // 16):
                    a = asg[pl.ds(v * 16, 16)]
                    b = adg[pl.ds(v * 16, 16)]
                    z = a + b
                    z = jnp.maximum(z, 0.2 * z)
                    eev[pl.ds(v * 16, 16)] = jnp.exp(z - m_s)

                def _scale(q, c2):
                    ev = eev[pl.ds(q * 16, 16)]
                    for l in range(16):
                        r = q * 16 + l
                        es = ev[l]
                        hrow[r, pl.ds(0, 16)] = hrow[r, pl.ds(0, 16)] * es
                        hrow[r, pl.ds(16, 16)] = hrow[r, pl.ds(16, 16)] * es
                    return c2
                lax.fori_loop(0, CH // 16, _scale, 0)

                for j in range(KROW):
                    pltpu.sync_copy(hrow.at[pl.ds(j * IDXW, IDXW)],
                                    acc_sh.at[dstv.at[j]], add=True)
                if do_den:
                    for j in range(KROW):
                        pltpu.sync_copy(eev.at[pl.ds(j * IDXW, IDXW)],
                                        den_sh.at[dstv.at[j]], add=True)
            return c

        lax.fori_loop(0, CPT, body, 0)

    @pl.when(cid == 0)
    def _():
        _mainloop(hA_hbm, True)

    @pl.when(cid == 1)
    def _():
        _mainloop(hB_hbm, False)

    plsc.subcore_barrier()

    pltpu.sync_copy(acc_sh.at[pl.ds(tid * ROWS_T, ROWS_T)], acc_out.at[cid, tid])

    @pl.when((cid == 0) & (tid < 2))
    def _():
        pltpu.sync_copy(den_sh.at[pl.ds(tid * DHALF, DHALF)], den_out.at[tid])


_sc_gat = pl.kernel(
    _sc_body,
    out_type=[
        jax.ShapeDtypeStruct((NSC, NT, ROWS_T, HH), jnp.float32),
        jax.ShapeDtypeStruct((2, DHALF), jnp.float32),
    ],
    mesh=plsc.VectorSubcoreMesh(core_axis_name="c", subcore_axis_name="s"),
    compiler_params=pltpu.CompilerParams(use_tc_tiling_on_sc=False),
    scratch_types=[
        pltpu.VMEM_SHARED((N, HH), jnp.float32),
        pltpu.VMEM_SHARED((N,), jnp.float32),
        pltpu.VMEM((KROW, IDXW), jnp.int32),
        pltpu.VMEM((KROW, IDXW), jnp.int32),
        pltpu.VMEM((CH,), jnp.float32),
        pltpu.VMEM((CH,), jnp.float32),
        pltpu.VMEM((CH,), jnp.float32),
        pltpu.VMEM((CH, HH), jnp.float32),
        pltpu.VMEM((1008,), jnp.float32),
        pltpu.VMEM((16,), jnp.float32),
        pltpu.SemaphoreType.DMA,
    ],
)


def _leaky_bound(m):
    z = m[0, 0] + m[0, 1]
    z = jnp.maximum(z, 0.2 * z)
    return jnp.broadcast_to(z, (16,))


@jax.jit
def kernel(x, edge_index, batch, W1, a1s, a1d, b1, W2, a2s, a2d, b2, Wl, bl):
    src = edge_index[0].reshape(NCHUNK, KROW, IDXW)
    dst = edge_index[1].reshape(NCHUNK, KROW, IDXW)

    A1 = jnp.stack([a1s, a1d], axis=1)
    A2 = jnp.stack([a2s, a2d], axis=1)

    h0, h1, aux, m = _k1(x, W1, A1)
    asv = aux[:, :, 0].reshape(N)
    adv = aux[:, :, 1].reshape(N)
    acc, den = _sc_gat(src, dst, asv, adv, h0, h1, _leaky_bound(m))
    accA = acc[0].reshape(N, HH)
    accB = acc[1].reshape(N, HH)
    den3 = den.reshape(NBLK, RB, 1)

    h0, h1, aux, m = _k2(accA, accB, den3, b1.reshape(1, H), W2, A2)
    asv = aux[:, :, 0].reshape(N)
    adv = aux[:, :, 1].reshape(N)
    acc, den = _sc_gat(src, dst, asv, adv, h0, h1, _leaky_bound(m))
    accA = acc[0].reshape(N, HH)
    accB = acc[1].reshape(N, HH)
    den3 = den.reshape(NBLK, RB, 1)

    batch3 = batch.reshape(NBLK, RB, 1)
    return _k3(accA, accB, den3, b2.reshape(1, H), batch3,
               Wl, bl.reshape(1, OUT))


# speedup vs baseline: 38.8353x; 1.2502x over previous
import functools

import jax
import jax.numpy as jnp
from jax import lax
from jax.experimental import pallas as pl
from jax.experimental.pallas import tpu as pltpu
from jax.experimental.pallas import tpu_sc as plsc

N = 50000
E = 800000
IN_DIM = 768
H = 64
HH = 32
G = 256
OUT = 2

RB = 2000
NBLK = N // RB

NSC = 2
NT = 16
IDXW = 128
KROW = 2
CH = IDXW * KROW
NCHUNK = E // CH
CPT = (NCHUNK + NT - 1) // NT
ROWS_T = N // NT
DHALF = N // 2

_EPS = 1e-16


def _tc_finish(h, A_ref, i, h0_ref, h1_ref, aux_ref, m_ref):
    aux = jnp.dot(h, A_ref[...], preferred_element_type=jnp.float32)
    h0_ref[...] = h[:, :HH]
    h1_ref[...] = h[:, HH:]
    aux_ref[...] = aux[None]
    mb = jnp.max(aux, axis=0)[None]

    @pl.when(i == 0)
    def _():
        m_ref[...] = mb

    @pl.when(i > 0)
    def _():
        m_ref[...] = jnp.maximum(m_ref[...], mb)


def _k1_body(x_ref, W_ref, A_ref, h0_ref, h1_ref, aux_ref, m_ref):
    i = pl.program_id(0)
    h = jnp.dot(x_ref[...], W_ref[...].T, preferred_element_type=jnp.float32)
    _tc_finish(h, A_ref, i, h0_ref, h1_ref, aux_ref, m_ref)


def _k2_body(accA_ref, accB_ref, den_ref, b_ref, W_ref, A_ref,
             h0_ref, h1_ref, aux_ref, m_ref):
    i = pl.program_id(0)
    inv = 1.0 / (den_ref[0] + _EPS)
    hin = jnp.concatenate([accA_ref[...], accB_ref[...]], axis=1)
    hin = jnp.maximum(hin * inv + b_ref[...], 0.0)
    h = jnp.dot(hin, W_ref[...].T, preferred_element_type=jnp.float32)
    _tc_finish(h, A_ref, i, h0_ref, h1_ref, aux_ref, m_ref)


def _k3_body(accA_ref, accB_ref, den_ref, b_ref, batch_ref, Wl_ref, bl_ref,
             out_ref, sums_ref):
    i = pl.program_id(0)
    inv = 1.0 / (den_ref[0] + _EPS)
    hin = jnp.concatenate([accA_ref[...], accB_ref[...]], axis=1)
    hin = jnp.maximum(hin * inv + b_ref[...], 0.0)
    hin = jnp.concatenate([hin, jnp.ones((RB, 1), jnp.float32)], axis=1)
    bb = batch_ref[0]
    gid = lax.broadcasted_iota(jnp.int32, (1, G), 1)
    onehot = (bb == gid).astype(jnp.float32)
    part = lax.dot_general(onehot, hin, (((0,), (0,)), ((), ())),
                           preferred_element_type=jnp.float32)

    @pl.when(i == 0)
    def _():
        sums_ref[...] = part

    @pl.when(i > 0)
    def _():
        sums_ref[...] = sums_ref[...] + part

    @pl.when(i == NBLK - 1)
    def _():
        s = sums_ref[...]
        g = s[:, :H] / jnp.maximum(s[:, H:], 1.0)
        out_ref[...] = (
            jnp.dot(g, Wl_ref[...].T, preferred_element_type=jnp.float32)
            + bl_ref[...])


_k1 = pl.pallas_call(
    _k1_body,
    grid=(NBLK,),
    in_specs=[
        pl.BlockSpec((RB, IN_DIM), lambda i: (i, 0)),
        pl.BlockSpec((H, IN_DIM), lambda i: (0, 0)),
        pl.BlockSpec((H, 2), lambda i: (0, 0)),
    ],
    out_specs=[
        pl.BlockSpec((RB, HH), lambda i: (i, 0)),
        pl.BlockSpec((RB, HH), lambda i: (i, 0)),
        pl.BlockSpec((1, RB, 2), lambda i: (i, 0, 0)),
        pl.BlockSpec((1, 2), lambda i: (0, 0)),
    ],
    out_shape=[
        jax.ShapeDtypeStruct((N, HH), jnp.float32),
        jax.ShapeDtypeStruct((N, HH), jnp.float32),
        jax.ShapeDtypeStruct((NBLK, RB, 2), jnp.float32),
        jax.ShapeDtypeStruct((1, 2), jnp.float32),
    ],
)

_k2 = pl.pallas_call(
    _k2_body,
    grid=(NBLK,),
    in_specs=[
        pl.BlockSpec((RB, HH), lambda i: (i, 0)),
        pl.BlockSpec((RB, HH), lambda i: (i, 0)),
        pl.BlockSpec((1, RB, 1), lambda i: (i, 0, 0)),
        pl.BlockSpec((1, H), lambda i: (0, 0)),
        pl.BlockSpec((H, H), lambda i: (0, 0)),
        pl.BlockSpec((H, 2), lambda i: (0, 0)),
    ],
    out_specs=[
        pl.BlockSpec((RB, HH), lambda i: (i, 0)),
        pl.BlockSpec((RB, HH), lambda i: (i, 0)),
        pl.BlockSpec((1, RB, 2), lambda i: (i, 0, 0)),
        pl.BlockSpec((1, 2), lambda i: (0, 0)),
    ],
    out_shape=[
        jax.ShapeDtypeStruct((N, HH), jnp.float32),
        jax.ShapeDtypeStruct((N, HH), jnp.float32),
        jax.ShapeDtypeStruct((NBLK, RB, 2), jnp.float32),
        jax.ShapeDtypeStruct((1, 2), jnp.float32),
    ],
)

_k3 = pl.pallas_call(
    _k3_body,
    grid=(NBLK,),
    in_specs=[
        pl.BlockSpec((RB, HH), lambda i: (i, 0)),
        pl.BlockSpec((RB, HH), lambda i: (i, 0)),
        pl.BlockSpec((1, RB, 1), lambda i: (i, 0, 0)),
        pl.BlockSpec((1, H), lambda i: (0, 0)),
        pl.BlockSpec((1, RB, 1), lambda i: (i, 0, 0)),
        pl.BlockSpec((OUT, H), lambda i: (0, 0)),
        pl.BlockSpec((1, OUT), lambda i: (0, 0)),
    ],
    out_specs=pl.BlockSpec((G, OUT), lambda i: (0, 0)),
    out_shape=jax.ShapeDtypeStruct((G, OUT), jnp.float32),
    scratch_shapes=[pltpu.VMEM((G, H + 1), jnp.float32)],
)


def _sc_body(src_hbm, dst_hbm, asv_hbm, adv_hbm, hA_hbm, hB_hbm, m_hbm,
             acc_out, den_out,
             acc_sh, den_sh,
             srcv0, dstv0, asg0, adg0, hrow0,
             srcv1, dstv1, asg1, adg1, hrow1,
             eev, zden, m_v, sem_i, sem_g):
    cid = lax.axis_index("c")
    tid = lax.axis_index("s")
    bufs = ((srcv0, dstv0, asg0, adg0, hrow0),
            (srcv1, dstv1, asg1, adg1, hrow1))

    def _zb(r, c):
        hrow0[r, pl.ds(0, 16)] = jnp.zeros((16,), jnp.float32)
        hrow0[r, pl.ds(16, 16)] = jnp.zeros((16,), jnp.float32)
        return c
    lax.fori_loop(0, CH, _zb, 0)

    def _zd(r, c):
        zden[pl.ds(r * 16, 16)] = jnp.zeros((16,), jnp.float32)
        return c
    lax.fori_loop(0, 1008 // 16, _zd, 0)

    for k in range(ROWS_T // CH):
        pltpu.sync_copy(hrow0, acc_sh.at[pl.ds(tid * ROWS_T + k * CH, CH)])
    _rem = ROWS_T - (ROWS_T // CH) * CH
    pltpu.sync_copy(hrow0.at[pl.ds(0, _rem)],
                    acc_sh.at[pl.ds(tid * ROWS_T + ROWS_T - _rem, _rem)])

    @pl.when(tid < 2)
    def _():
        def _zdn(k, c):
            pltpu.sync_copy(zden.at[pl.ds(0, 1000)],
                            den_sh.at[pl.ds(tid * DHALF + k * 1000, 1000)])
            return c
        lax.fori_loop(0, DHALF // 1000, _zdn, 0)

    pltpu.sync_copy(m_hbm, m_v)
    plsc.subcore_barrier()

    m_s = m_v[pl.ds(0, 16)][0]

    def _fire_srcdst(sid, b):
        pltpu.async_copy(src_hbm.at[sid], bufs[b][0], sem_i)
        pltpu.async_copy(dst_hbm.at[sid], bufs[b][1], sem_i)

    def _wait_srcdst(b):
        pltpu.make_async_copy(src_hbm.at[0], bufs[b][0], sem_i).wait()
        pltpu.make_async_copy(dst_hbm.at[0], bufs[b][1], sem_i).wait()

    def _mainloop(htab, do_den):
        base = tid * CPT

        def _fire_gathers(b):
            srcv, dstv, asg, adg, hrow = bufs[b]
            for j in range(KROW):
                pltpu.async_copy(asv_hbm.at[srcv.at[j]],
                                 asg.at[pl.ds(j * IDXW, IDXW)], sem_g)
                pltpu.async_copy(adv_hbm.at[dstv.at[j]],
                                 adg.at[pl.ds(j * IDXW, IDXW)], sem_g)
                pltpu.async_copy(htab.at[srcv.at[j]],
                                 hrow.at[pl.ds(j * IDXW, IDXW)], sem_g)

        def _wait_gathers(b):
            srcv, dstv, asg, adg, hrow = bufs[b]
            for j in range(KROW):
                pltpu.make_async_copy(asv_hbm.at[srcv.at[j]],
                                      asg.at[pl.ds(j * IDXW, IDXW)],
                                      sem_g).wait()
                pltpu.make_async_copy(adv_hbm.at[dstv.at[j]],
                                      adg.at[pl.ds(j * IDXW, IDXW)],
                                      sem_g).wait()
                pltpu.make_async_copy(htab.at[srcv.at[j]],
                                      hrow.at[pl.ds(j * IDXW, IDXW)],
                                      sem_g).wait()

        def _step(k, b):
            sid = base + k
            _wait_srcdst(1 - b)
            _fire_gathers(1 - b)
            _wait_gathers(b)

            srcv, dstv, asg, adg, hrow = bufs[b]
            for v in range(CH // 16):
                a = asg[pl.ds(v * 16, 16)]
                bb = adg[pl.ds(v * 16, 16)]
                z = a + bb
                z = jnp.maximum(z, 0.2 * z)
                eev[pl.ds(v * 16, 16)] = jnp.exp(z - m_s)

            def _scale(q, c2):
                ev = eev[pl.ds(q * 16, 16)]
                for l in range(16):
                    r = q * 16 + l
                    es = ev[l]
                    hrow[r, pl.ds(0, 16)] = hrow[r, pl.ds(0, 16)] * es
                    hrow[r, pl.ds(16, 16)] = hrow[r, pl.ds(16, 16)] * es
                return c2
            lax.fori_loop(0, CH // 16, _scale, 0)

            @pl.when(sid < NCHUNK)
            def _():
                for j in range(KROW):
                    pltpu.sync_copy(hrow.at[pl.ds(j * IDXW, IDXW)],
                                    acc_sh.at[dstv.at[j]], add=True)
                if do_den:
                    for j in range(KROW):
                        pltpu.sync_copy(eev.at[pl.ds(j * IDXW, IDXW)],
                                        den_sh.at[dstv.at[j]], add=True)

            _fire_srcdst(jnp.minimum(sid + 2, NCHUNK - 1), b)

        _fire_srcdst(base, 0)
        _fire_srcdst(jnp.minimum(base + 1, NCHUNK - 1), 1)
        _wait_srcdst(0)
        _fire_gathers(0)

        def _pair(i, c):
            _step(2 * i, 0)
            _step(2 * i + 1, 1)
            return c
        lax.fori_loop(0, CPT // 2, _pair, 0)

        _wait_srcdst(1)
        _wait_gathers(CPT % 2)

    @pl.when(cid == 0)
    def _():
        _mainloop(hA_hbm, True)

    @pl.when(cid == 1)
    def _():
        _mainloop(hB_hbm, False)

    plsc.subcore_barrier()

    pltpu.sync_copy(acc_sh.at[pl.ds(tid * ROWS_T, ROWS_T)],
                    acc_out.at[cid, pl.ds(tid * ROWS_T, ROWS_T)])

    @pl.when((cid == 0) & (tid < 2))
    def _():
        pltpu.sync_copy(den_sh.at[pl.ds(tid * DHALF, DHALF)],
                        den_out.at[pl.ds(tid * DHALF, DHALF)])


_sc_gat = pl.kernel(
    _sc_body,
    out_type=[
        jax.ShapeDtypeStruct((NSC, N, HH), jnp.float32),
        jax.ShapeDtypeStruct((N,), jnp.float32),
    ],
    mesh=plsc.VectorSubcoreMesh(core_axis_name="c", subcore_axis_name="s"),
    compiler_params=pltpu.CompilerParams(use_tc_tiling_on_sc=False),
    scratch_types=(
        [pltpu.VMEM_SHARED((N, HH), jnp.float32),
         pltpu.VMEM_SHARED((N,), jnp.float32)]
        + 2 * [
            pltpu.VMEM((KROW, IDXW), jnp.int32),
            pltpu.VMEM((KROW, IDXW), jnp.int32),
            pltpu.VMEM((CH,), jnp.float32),
            pltpu.VMEM((CH,), jnp.float32),
            pltpu.VMEM((CH, HH), jnp.float32),
        ]
        + [
            pltpu.VMEM((CH,), jnp.float32),
            pltpu.VMEM((1008,), jnp.float32),
            pltpu.VMEM((16,), jnp.float32),
            pltpu.SemaphoreType.DMA,
            pltpu.SemaphoreType.DMA,
        ]
    ),
)


def _leaky_bound(m):
    z = m[0, 0] + m[0, 1]
    z = jnp.maximum(z, 0.2 * z)
    return jnp.broadcast_to(z, (16,))


@jax.jit
def kernel(x, edge_index, batch, W1, a1s, a1d, b1, W2, a2s, a2d, b2, Wl, bl):
    src = edge_index[0].reshape(NCHUNK, KROW, IDXW)
    dst = edge_index[1].reshape(NCHUNK, KROW, IDXW)

    A1 = jnp.stack([a1s, a1d], axis=1)
    A2 = jnp.stack([a2s, a2d], axis=1)

    h0, h1, aux, m = _k1(x, W1, A1)
    asv = aux[:, :, 0].reshape(N)
    adv = aux[:, :, 1].reshape(N)
    acc, den = _sc_gat(src, dst, asv, adv, h0, h1, _leaky_bound(m))
    den3 = den.reshape(NBLK, RB, 1)

    h0, h1, aux, m = _k2(acc[0], acc[1], den3, b1.reshape(1, H), W2, A2)
    asv = aux[:, :, 0].reshape(N)
    adv = aux[:, :, 1].reshape(N)
    acc, den = _sc_gat(src, dst, asv, adv, h0, h1, _leaky_bound(m))
    den3 = den.reshape(NBLK, RB, 1)

    batch3 = batch.reshape(NBLK, RB, 1)
    return _k3(acc[0], acc[1], den3, b2.reshape(1, H), batch3,
               Wl, bl.reshape(1, OUT))


# speedup vs baseline: 42.4305x; 1.0926x over previous
import functools

import jax
import jax.numpy as jnp
from jax import lax
from jax.experimental import pallas as pl
from jax.experimental.pallas import tpu as pltpu
from jax.experimental.pallas import tpu_sc as plsc

N = 50000
E = 800000
IN_DIM = 768
H = 64
HH = 32
G = 256
OUT = 2

RB = 2000
NBLK = N // RB

NSC = 2
NT = 16
IDXW = 128
KROW = 2
CH = IDXW * KROW
NCHUNK = E // CH
CPT = (NCHUNK + NT - 1) // NT
ROWS_T = N // NT
DHALF = N // 2

_EPS = 1e-16


def _tc_finish(h, A_ref, i, h0_ref, h1_ref, aux_ref, m_ref):
    aux = jnp.dot(h, A_ref[...], preferred_element_type=jnp.float32)
    h0_ref[...] = h[:, :HH]
    h1_ref[...] = h[:, HH:]
    aux_ref[...] = aux[None]
    mb = jnp.max(aux, axis=0)[None]

    @pl.when(i == 0)
    def _():
        m_ref[...] = mb

    @pl.when(i > 0)
    def _():
        m_ref[...] = jnp.maximum(m_ref[...], mb)


def _k1_body(x_ref, W_ref, A_ref, h0_ref, h1_ref, aux_ref, m_ref):
    i = pl.program_id(0)
    h = jnp.dot(x_ref[...], W_ref[...].T, preferred_element_type=jnp.float32)
    _tc_finish(h, A_ref, i, h0_ref, h1_ref, aux_ref, m_ref)


def _k2_body(accA_ref, accB_ref, den_ref, b_ref, W_ref, A_ref,
             h0_ref, h1_ref, aux_ref, m_ref):
    i = pl.program_id(0)
    inv = 1.0 / (den_ref[0] + _EPS)
    hin = jnp.concatenate([accA_ref[...], accB_ref[...]], axis=1)
    hin = jnp.maximum(hin * inv + b_ref[...], 0.0)
    h = jnp.dot(hin, W_ref[...].T, preferred_element_type=jnp.float32)
    _tc_finish(h, A_ref, i, h0_ref, h1_ref, aux_ref, m_ref)


def _k3_body(accA_ref, accB_ref, den_ref, b_ref, batch_ref, Wl_ref, bl_ref,
             out_ref, sums_ref):
    i = pl.program_id(0)
    inv = 1.0 / (den_ref[0] + _EPS)
    hin = jnp.concatenate([accA_ref[...], accB_ref[...]], axis=1)
    hin = jnp.maximum(hin * inv + b_ref[...], 0.0)
    hin = jnp.concatenate([hin, jnp.ones((RB, 1), jnp.float32)], axis=1)
    bb = batch_ref[0]
    gid = lax.broadcasted_iota(jnp.int32, (1, G), 1)
    onehot = (bb == gid).astype(jnp.float32)
    part = lax.dot_general(onehot, hin, (((0,), (0,)), ((), ())),
                           preferred_element_type=jnp.float32)

    @pl.when(i == 0)
    def _():
        sums_ref[...] = part

    @pl.when(i > 0)
    def _():
        sums_ref[...] = sums_ref[...] + part

    @pl.when(i == NBLK - 1)
    def _():
        s = sums_ref[...]
        g = s[:, :H] / jnp.maximum(s[:, H:], 1.0)
        out_ref[...] = (
            jnp.dot(g, Wl_ref[...].T, preferred_element_type=jnp.float32)
            + bl_ref[...])


_k1 = pl.pallas_call(
    _k1_body,
    grid=(NBLK,),
    in_specs=[
        pl.BlockSpec((RB, IN_DIM), lambda i: (i, 0)),
        pl.BlockSpec((H, IN_DIM), lambda i: (0, 0)),
        pl.BlockSpec((H, 2), lambda i: (0, 0)),
    ],
    out_specs=[
        pl.BlockSpec((RB, HH), lambda i: (i, 0)),
        pl.BlockSpec((RB, HH), lambda i: (i, 0)),
        pl.BlockSpec((1, RB, 2), lambda i: (i, 0, 0)),
        pl.BlockSpec((1, 2), lambda i: (0, 0)),
    ],
    out_shape=[
        jax.ShapeDtypeStruct((N, HH), jnp.float32),
        jax.ShapeDtypeStruct((N, HH), jnp.float32),
        jax.ShapeDtypeStruct((NBLK, RB, 2), jnp.float32),
        jax.ShapeDtypeStruct((1, 2), jnp.float32),
    ],
)

_k2 = pl.pallas_call(
    _k2_body,
    grid=(NBLK,),
    in_specs=[
        pl.BlockSpec((RB, HH), lambda i: (i, 0)),
        pl.BlockSpec((RB, HH), lambda i: (i, 0)),
        pl.BlockSpec((1, RB, 1), lambda i: (i, 0, 0)),
        pl.BlockSpec((1, H), lambda i: (0, 0)),
        pl.BlockSpec((H, H), lambda i: (0, 0)),
        pl.BlockSpec((H, 2), lambda i: (0, 0)),
    ],
    out_specs=[
        pl.BlockSpec((RB, HH), lambda i: (i, 0)),
        pl.BlockSpec((RB, HH), lambda i: (i, 0)),
        pl.BlockSpec((1, RB, 2), lambda i: (i, 0, 0)),
        pl.BlockSpec((1, 2), lambda i: (0, 0)),
    ],
    out_shape=[
        jax.ShapeDtypeStruct((N, HH), jnp.float32),
        jax.ShapeDtypeStruct((N, HH), jnp.float32),
        jax.ShapeDtypeStruct((NBLK, RB, 2), jnp.float32),
        jax.ShapeDtypeStruct((1, 2), jnp.float32),
    ],
)

_k3 = pl.pallas_call(
    _k3_body,
    grid=(NBLK,),
    in_specs=[
        pl.BlockSpec((RB, HH), lambda i: (i, 0)),
        pl.BlockSpec((RB, HH), lambda i: (i, 0)),
        pl.BlockSpec((1, RB, 1), lambda i: (i, 0, 0)),
        pl.BlockSpec((1, H), lambda i: (0, 0)),
        pl.BlockSpec((1, RB, 1), lambda i: (i, 0, 0)),
        pl.BlockSpec((OUT, H), lambda i: (0, 0)),
        pl.BlockSpec((1, OUT), lambda i: (0, 0)),
    ],
    out_specs=pl.BlockSpec((G, OUT), lambda i: (0, 0)),
    out_shape=jax.ShapeDtypeStruct((G, OUT), jnp.float32),
    scratch_shapes=[pltpu.VMEM((G, H + 1), jnp.float32)],
)


def _sc_body(src_hbm, dst_hbm, asv_hbm, adv_hbm, hA_hbm, hB_hbm, m_hbm,
             acc_out, den_out,
             acc_sh, den_sh,
             srcv0, dstv0, asg0, adg0, hrow0, dstc0, eev0,
             srcv1, dstv1, asg1, adg1, hrow1, dstc1, eev1,
             zden, m_v, sem_i, sem_g, sem_s):
    cid = lax.axis_index("c")
    tid = lax.axis_index("s")
    bufs = ((srcv0, dstv0, asg0, adg0, hrow0, dstc0, eev0),
            (srcv1, dstv1, asg1, adg1, hrow1, dstc1, eev1))

    def _zb(r, c):
        hrow0[r, pl.ds(0, 16)] = jnp.zeros((16,), jnp.float32)
        hrow0[r, pl.ds(16, 16)] = jnp.zeros((16,), jnp.float32)
        return c
    lax.fori_loop(0, CH, _zb, 0)

    def _zd(r, c):
        zden[pl.ds(r * 16, 16)] = jnp.zeros((16,), jnp.float32)
        return c
    lax.fori_loop(0, 1008 // 16, _zd, 0)

    for k in range(ROWS_T // CH):
        pltpu.sync_copy(hrow0, acc_sh.at[pl.ds(tid * ROWS_T + k * CH, CH)])
    _rem = ROWS_T - (ROWS_T // CH) * CH
    pltpu.sync_copy(hrow0.at[pl.ds(0, _rem)],
                    acc_sh.at[pl.ds(tid * ROWS_T + ROWS_T - _rem, _rem)])

    @pl.when(tid < 2)
    def _():
        def _zdn(k, c):
            pltpu.sync_copy(zden.at[pl.ds(0, 1000)],
                            den_sh.at[pl.ds(tid * DHALF + k * 1000, 1000)])
            return c
        lax.fori_loop(0, DHALF // 1000, _zdn, 0)

    pltpu.sync_copy(m_hbm, m_v)
    plsc.subcore_barrier()

    m_s = m_v[pl.ds(0, 16)][0]

    def _fire_srcdst(sid, b):
        pltpu.async_copy(src_hbm.at[sid], bufs[b][0], sem_i)
        pltpu.async_copy(dst_hbm.at[sid], bufs[b][1], sem_i)

    def _wait_srcdst(b):
        pltpu.make_async_copy(src_hbm.at[0], bufs[b][0], sem_i).wait()
        pltpu.make_async_copy(dst_hbm.at[0], bufs[b][1], sem_i).wait()

    def _mainloop(htab, do_den):
        base = tid * CPT

        def _fire_gathers(b):
            srcv, dstv, asg, adg, hrow = bufs[b][:5]
            for j in range(KROW):
                pltpu.async_copy(asv_hbm.at[srcv.at[j]],
                                 asg.at[pl.ds(j * IDXW, IDXW)], sem_g)
                pltpu.async_copy(adv_hbm.at[dstv.at[j]],
                                 adg.at[pl.ds(j * IDXW, IDXW)], sem_g)
                pltpu.async_copy(htab.at[srcv.at[j]],
                                 hrow.at[pl.ds(j * IDXW, IDXW)], sem_g)

        def _wait_gathers(b):
            srcv, dstv, asg, adg, hrow = bufs[b][:5]
            for j in range(KROW):
                pltpu.make_async_copy(asv_hbm.at[srcv.at[j]],
                                      asg.at[pl.ds(j * IDXW, IDXW)],
                                      sem_g).wait()
                pltpu.make_async_copy(adv_hbm.at[dstv.at[j]],
                                      adg.at[pl.ds(j * IDXW, IDXW)],
                                      sem_g).wait()
                pltpu.make_async_copy(htab.at[srcv.at[j]],
                                      hrow.at[pl.ds(j * IDXW, IDXW)],
                                      sem_g).wait()

        def _fire_scatters(b):
            hrow, dstc, eev = bufs[b][4:7]
            for j in range(KROW):
                pltpu.async_copy(hrow.at[pl.ds(j * IDXW, IDXW)],
                                 acc_sh.at[dstc.at[j]], sem_s, add=True)
            if do_den:
                for j in range(KROW):
                    pltpu.async_copy(eev.at[pl.ds(j * IDXW, IDXW)],
                                     den_sh.at[dstc.at[j]], sem_s, add=True)

        def _wait_scatters(b):
            hrow, dstc, eev = bufs[b][4:7]
            for j in range(KROW):
                pltpu.make_async_copy(hrow.at[pl.ds(j * IDXW, IDXW)],
                                      acc_sh.at[dstc.at[j]], sem_s).wait()
            if do_den:
                for j in range(KROW):
                    pltpu.make_async_copy(eev.at[pl.ds(j * IDXW, IDXW)],
                                          den_sh.at[dstc.at[j]], sem_s).wait()

        def _step(k, b):
            sid = base + k
            _wait_srcdst(1 - b)

            @pl.when(k > 0)
            def _():
                _wait_scatters(1 - b)

            _fire_gathers(1 - b)
            _wait_gathers(b)

            srcv, dstv, asg, adg, hrow, dstc, eev = bufs[b]
            gate = jnp.where(sid < NCHUNK,
                             jnp.full((16,), 1.0, jnp.float32),
                             jnp.full((16,), 0.0, jnp.float32))

            for v in range(CH // 16):
                a = asg[pl.ds(v * 16, 16)]
                bb = adg[pl.ds(v * 16, 16)]
                z = a + bb
                z = jnp.maximum(z, 0.2 * z)
                eev[pl.ds(v * 16, 16)] = jnp.exp(z - m_s) * gate

            def _scale(q, c2):
                ev = eev[pl.ds(q * 16, 16)]
                for l in range(16):
                    r = q * 16 + l
                    es = ev[l]
                    hrow[r, pl.ds(0, 16)] = hrow[r, pl.ds(0, 16)] * es
                    hrow[r, pl.ds(16, 16)] = hrow[r, pl.ds(16, 16)] * es
                return c2
            lax.fori_loop(0, CH // 16, _scale, 0)

            for j in range(KROW):
                for v in range(IDXW // 16):
                    dstc[j, pl.ds(v * 16, 16)] = dstv[j, pl.ds(v * 16, 16)]

            _fire_scatters(b)
            _fire_srcdst(jnp.minimum(sid + 2, NCHUNK - 1), b)

        _fire_srcdst(base, 0)
        _fire_srcdst(jnp.minimum(base + 1, NCHUNK - 1), 1)
        _wait_srcdst(0)
        _fire_gathers(0)

        def _pair(i, c):
            _step(2 * i, 0)
            _step(2 * i + 1, 1)
            return c
        lax.fori_loop(0, CPT // 2, _pair, 0)

        _wait_srcdst(1)
        _wait_gathers(CPT % 2)
        _wait_scatters(1 - CPT % 2)

    @pl.when(cid == 0)
    def _():
        _mainloop(hA_hbm, True)

    @pl.when(cid == 1)
    def _():
        _mainloop(hB_hbm, False)

    plsc.subcore_barrier()

    pltpu.sync_copy(acc_sh.at[pl.ds(tid * ROWS_T, ROWS_T)],
                    acc_out.at[cid, pl.ds(tid * ROWS_T, ROWS_T)])

    @pl.when((cid == 0) & (tid < 2))
    def _():
        pltpu.sync_copy(den_sh.at[pl.ds(tid * DHALF, DHALF)],
                        den_out.at[pl.ds(tid * DHALF, DHALF)])


_sc_gat = pl.kernel(
    _sc_body,
    out_type=[
        jax.ShapeDtypeStruct((NSC, N, HH), jnp.float32),
        jax.ShapeDtypeStruct((N,), jnp.float32),
    ],
    mesh=plsc.VectorSubcoreMesh(core_axis_name="c", subcore_axis_name="s"),
    compiler_params=pltpu.CompilerParams(use_tc_tiling_on_sc=False),
    scratch_types=(
        [pltpu.VMEM_SHARED((N, HH), jnp.float32),
         pltpu.VMEM_SHARED((N,), jnp.float32)]
        + 2 * [
            pltpu.VMEM((KROW, IDXW), jnp.int32),
            pltpu.VMEM((KROW, IDXW), jnp.int32),
            pltpu.VMEM((CH,), jnp.float32),
            pltpu.VMEM((CH,), jnp.float32),
            pltpu.VMEM((CH, HH), jnp.float32),
            pltpu.VMEM((KROW, IDXW), jnp.int32),
            pltpu.VMEM((CH,), jnp.float32),
        ]
        + [
            pltpu.VMEM((1008,), jnp.float32),
            pltpu.VMEM((16,), jnp.float32),
            pltpu.SemaphoreType.DMA,
            pltpu.SemaphoreType.DMA,
            pltpu.SemaphoreType.DMA,
        ]
    ),
)


def _leaky_bound(m):
    z = m[0, 0] + m[0, 1]
    z = jnp.maximum(z, 0.2 * z)
    return jnp.broadcast_to(z, (16,))


@jax.jit
def kernel(x, edge_index, batch, W1, a1s, a1d, b1, W2, a2s, a2d, b2, Wl, bl):
    src = edge_index[0].reshape(NCHUNK, KROW, IDXW)
    dst = edge_index[1].reshape(NCHUNK, KROW, IDXW)

    A1 = jnp.stack([a1s, a1d], axis=1)
    A2 = jnp.stack([a2s, a2d], axis=1)

    h0, h1, aux, m = _k1(x, W1, A1)
    asv = aux[:, :, 0].reshape(N)
    adv = aux[:, :, 1].reshape(N)
    acc, den = _sc_gat(src, dst, asv, adv, h0, h1, _leaky_bound(m))
    den3 = den.reshape(NBLK, RB, 1)

    h0, h1, aux, m = _k2(acc[0], acc[1], den3, b1.reshape(1, H), W2, A2)
    asv = aux[:, :, 0].reshape(N)
    adv = aux[:, :, 1].reshape(N)
    acc, den = _sc_gat(src, dst, asv, adv, h0, h1, _leaky_bound(m))
    den3 = den.reshape(NBLK, RB, 1)

    batch3 = batch.reshape(NBLK, RB, 1)
    return _k3(acc[0], acc[1], den3, b2.reshape(1, H), batch3,
               Wl, bl.reshape(1, OUT))


# speedup vs baseline: 53.6915x; 1.2654x over previous
import functools

import jax
import jax.numpy as jnp
from jax import lax
from jax.experimental import pallas as pl
from jax.experimental.pallas import tpu as pltpu
from jax.experimental.pallas import tpu_sc as plsc

N = 50000
E = 800000
IN_DIM = 768
H = 64
HH = 32
G = 256
OUT = 2

RB = 2000
NBLK = N // RB

NSC = 2
NT = 16
IDXW = 128
KROW = 2
CH = IDXW * KROW
NCHUNK = E // CH
CPT = (NCHUNK + NT - 1) // NT
ROWS_T = N // NT
DHALF = N // 2

_EPS = 1e-16


def _tc_finish(h, A_ref, i, h0_ref, h1_ref, as_ref, ad_ref, m_ref):
    aux = jnp.dot(h, A_ref[...], preferred_element_type=jnp.float32)
    h0_ref[...] = h[:, :HH]
    h1_ref[...] = h[:, HH:]
    as_ref[...] = aux[:, 0:1].T[None]
    ad_ref[...] = aux[:, 1:2].T[None]
    mb = jnp.max(aux, axis=0)[None]

    @pl.when(i == 0)
    def _():
        m_ref[...] = mb

    @pl.when(i > 0)
    def _():
        m_ref[...] = jnp.maximum(m_ref[...], mb)


def _k1_body(x_ref, W_ref, A_ref, h0_ref, h1_ref, as_ref, ad_ref, m_ref):
    i = pl.program_id(0)
    h = jnp.dot(x_ref[...], W_ref[...].T, preferred_element_type=jnp.float32)
    _tc_finish(h, A_ref, i, h0_ref, h1_ref, as_ref, ad_ref, m_ref)


def _norm_in(accd_ref, den_ref, b_ref):
    inv = jnp.reshape(1.0 / (den_ref[0] + _EPS), (RB, 1))
    hin = accd_ref[...][:, :H]
    return jnp.maximum(hin * inv + b_ref[...], 0.0)


def _k2_body(accd_ref, den_ref, b_ref, W_ref, A_ref,
             h0_ref, h1_ref, as_ref, ad_ref, m_ref):
    i = pl.program_id(0)
    hin = _norm_in(accd_ref, den_ref, b_ref)
    h = jnp.dot(hin, W_ref[...].T, preferred_element_type=jnp.float32)
    _tc_finish(h, A_ref, i, h0_ref, h1_ref, as_ref, ad_ref, m_ref)


def _k3_body(accd_ref, den_ref, b_ref, batch_ref, Wl_ref, bl_ref,
             out_ref, sums_ref):
    i = pl.program_id(0)
    hin = _norm_in(accd_ref, den_ref, b_ref)
    hin = jnp.concatenate([hin, jnp.ones((RB, 1), jnp.float32)], axis=1)
    bb = jnp.reshape(batch_ref[0], (RB, 1))
    gid = lax.broadcasted_iota(jnp.int32, (1, G), 1)
    onehot = (bb == gid).astype(jnp.float32)
    part = lax.dot_general(onehot, hin, (((0,), (0,)), ((), ())),
                           preferred_element_type=jnp.float32)

    @pl.when(i == 0)
    def _():
        sums_ref[...] = part

    @pl.when(i > 0)
    def _():
        sums_ref[...] = sums_ref[...] + part

    @pl.when(i == NBLK - 1)
    def _():
        s = sums_ref[...]
        g = s[:, :H] / jnp.maximum(s[:, H:], 1.0)
        out_ref[...] = (
            jnp.dot(g, Wl_ref[...].T, preferred_element_type=jnp.float32)
            + bl_ref[...])


_HEAD_OUT_SPECS = [
    pl.BlockSpec((RB, HH), lambda i: (i, 0)),
    pl.BlockSpec((RB, HH), lambda i: (i, 0)),
    pl.BlockSpec((1, 1, RB), lambda i: (i, 0, 0)),
    pl.BlockSpec((1, 1, RB), lambda i: (i, 0, 0)),
    pl.BlockSpec((1, 2), lambda i: (0, 0)),
]
_HEAD_OUT_SHAPE = [
    jax.ShapeDtypeStruct((N, HH), jnp.float32),
    jax.ShapeDtypeStruct((N, HH), jnp.float32),
    jax.ShapeDtypeStruct((NBLK, 1, RB), jnp.float32),
    jax.ShapeDtypeStruct((NBLK, 1, RB), jnp.float32),
    jax.ShapeDtypeStruct((1, 2), jnp.float32),
]

_k1 = pl.pallas_call(
    _k1_body,
    grid=(NBLK,),
    in_specs=[
        pl.BlockSpec((RB, IN_DIM), lambda i: (i, 0)),
        pl.BlockSpec((H, IN_DIM), lambda i: (0, 0)),
        pl.BlockSpec((H, 2), lambda i: (0, 0)),
    ],
    out_specs=_HEAD_OUT_SPECS,
    out_shape=_HEAD_OUT_SHAPE,
)

_k2 = pl.pallas_call(
    _k2_body,
    grid=(NBLK,),
    in_specs=[
        pl.BlockSpec((RB, 128), lambda i: (i, 0)),
        pl.BlockSpec((1, 1, RB), lambda i: (i, 0, 0)),
        pl.BlockSpec((1, H), lambda i: (0, 0)),
        pl.BlockSpec((H, H), lambda i: (0, 0)),
        pl.BlockSpec((H, 2), lambda i: (0, 0)),
    ],
    out_specs=_HEAD_OUT_SPECS,
    out_shape=_HEAD_OUT_SHAPE,
)

_k3 = pl.pallas_call(
    _k3_body,
    grid=(NBLK,),
    in_specs=[
        pl.BlockSpec((RB, 128), lambda i: (i, 0)),
        pl.BlockSpec((1, 1, RB), lambda i: (i, 0, 0)),
        pl.BlockSpec((1, H), lambda i: (0, 0)),
        pl.BlockSpec((1, 1, RB), lambda i: (i, 0, 0)),
        pl.BlockSpec((OUT, H), lambda i: (0, 0)),
        pl.BlockSpec((1, OUT), lambda i: (0, 0)),
    ],
    out_specs=pl.BlockSpec((G, OUT), lambda i: (0, 0)),
    out_shape=jax.ShapeDtypeStruct((G, OUT), jnp.float32),
    scratch_shapes=[pltpu.VMEM((G, H + 1), jnp.float32)],
)


def _sc_body(src_hbm, dst_hbm, asv_hbm, adv_hbm, hA_hbm, hB_hbm, m_hbm,
             acc_out, den_out,
             acc_sh, den_sh,
             srcv0, dstv0, asg0, adg0, hrow0, dstc0, eev0,
             srcv1, dstv1, asg1, adg1, hrow1, dstc1, eev1,
             zden, m_v, sem_i, sem_g, sem_s):
    cid = lax.axis_index("c")
    tid = lax.axis_index("s")
    bufs = ((srcv0, dstv0, asg0, adg0, hrow0, dstc0, eev0),
            (srcv1, dstv1, asg1, adg1, hrow1, dstc1, eev1))

    def _zb(r, c):
        hrow0[r, pl.ds(0, 16)] = jnp.zeros((16,), jnp.float32)
        hrow0[r, pl.ds(16, 16)] = jnp.zeros((16,), jnp.float32)
        return c
    lax.fori_loop(0, CH, _zb, 0)

    def _zd(r, c):
        zden[pl.ds(r * 16, 16)] = jnp.zeros((16,), jnp.float32)
        return c
    lax.fori_loop(0, 1008 // 16, _zd, 0)

    for k in range(ROWS_T // CH):
        pltpu.sync_copy(hrow0, acc_sh.at[pl.ds(tid * ROWS_T + k * CH, CH)])
    _rem = ROWS_T - (ROWS_T // CH) * CH
    pltpu.sync_copy(hrow0.at[pl.ds(0, _rem)],
                    acc_sh.at[pl.ds(tid * ROWS_T + ROWS_T - _rem, _rem)])

    @pl.when(tid < 2)
    def _():
        def _zdn(k, c):
            pltpu.sync_copy(zden.at[pl.ds(0, 1000)],
                            den_sh.at[pl.ds(tid * DHALF + k * 1000, 1000)])
            return c
        lax.fori_loop(0, DHALF // 1000, _zdn, 0)

    pltpu.sync_copy(m_hbm, m_v)
    plsc.subcore_barrier()

    m_s = m_v[pl.ds(0, 16)][0]

    def _fire_srcdst(sid, b):
        pltpu.async_copy(src_hbm.at[sid], bufs[b][0], sem_i)
        pltpu.async_copy(dst_hbm.at[sid], bufs[b][1], sem_i)

    def _wait_srcdst(b):
        pltpu.make_async_copy(src_hbm.at[0], bufs[b][0], sem_i).wait()
        pltpu.make_async_copy(dst_hbm.at[0], bufs[b][1], sem_i).wait()

    def _mainloop(htab, do_den):
        base = tid * CPT

        def _fire_gathers(b):
            srcv, dstv, asg, adg, hrow = bufs[b][:5]
            for j in range(KROW):
                pltpu.async_copy(asv_hbm.at[srcv.at[j]],
                                 asg.at[pl.ds(j * IDXW, IDXW)], sem_g)
                pltpu.async_copy(adv_hbm.at[dstv.at[j]],
                                 adg.at[pl.ds(j * IDXW, IDXW)], sem_g)
                pltpu.async_copy(htab.at[srcv.at[j]],
                                 hrow.at[pl.ds(j * IDXW, IDXW)], sem_g)

        def _wait_gathers(b):
            srcv, dstv, asg, adg, hrow = bufs[b][:5]
            for j in range(KROW):
                pltpu.make_async_copy(asv_hbm.at[srcv.at[j]],
                                      asg.at[pl.ds(j * IDXW, IDXW)],
                                      sem_g).wait()
                pltpu.make_async_copy(adv_hbm.at[dstv.at[j]],
                                      adg.at[pl.ds(j * IDXW, IDXW)],
                                      sem_g).wait()
                pltpu.make_async_copy(htab.at[srcv.at[j]],
                                      hrow.at[pl.ds(j * IDXW, IDXW)],
                                      sem_g).wait()

        def _fire_scatters(b):
            hrow, dstc, eev = bufs[b][4:7]
            for j in range(KROW):
                pltpu.async_copy(hrow.at[pl.ds(j * IDXW, IDXW)],
                                 acc_sh.at[dstc.at[j]], sem_s, add=True)
            if do_den:
                for j in range(KROW):
                    pltpu.async_copy(eev.at[pl.ds(j * IDXW, IDXW)],
                                     den_sh.at[dstc.at[j]], sem_s, add=True)

        def _wait_scatters(b):
            hrow, dstc, eev = bufs[b][4:7]
            for j in range(KROW):
                pltpu.make_async_copy(hrow.at[pl.ds(j * IDXW, IDXW)],
                                      acc_sh.at[dstc.at[j]], sem_s).wait()
            if do_den:
                for j in range(KROW):
                    pltpu.make_async_copy(eev.at[pl.ds(j * IDXW, IDXW)],
                                          den_sh.at[dstc.at[j]], sem_s).wait()

        def _step(k, b):
            sid = base + k
            _wait_srcdst(1 - b)

            @pl.when(k > 0)
            def _():
                _wait_scatters(1 - b)

            _fire_gathers(1 - b)
            _wait_gathers(b)

            srcv, dstv, asg, adg, hrow, dstc, eev = bufs[b]
            gate = jnp.where(sid < NCHUNK,
                             jnp.full((16,), 1.0, jnp.float32),
                             jnp.full((16,), 0.0, jnp.float32))

            for v in range(CH // 16):
                a = asg[pl.ds(v * 16, 16)]
                bb = adg[pl.ds(v * 16, 16)]
                z = a + bb
                z = jnp.maximum(z, 0.2 * z)
                eev[pl.ds(v * 16, 16)] = jnp.exp(z - m_s) * gate

            def _scale(q, c2):
                ev = eev[pl.ds(q * 16, 16)]
                for l in range(16):
                    r = q * 16 + l
                    es = ev[l]
                    hrow[r, pl.ds(0, 16)] = hrow[r, pl.ds(0, 16)] * es
                    hrow[r, pl.ds(16, 16)] = hrow[r, pl.ds(16, 16)] * es
                return c2
            lax.fori_loop(0, CH // 16, _scale, 0)

            for j in range(KROW):
                for v in range(IDXW // 16):
                    dstc[j, pl.ds(v * 16, 16)] = dstv[j, pl.ds(v * 16, 16)]

            _fire_scatters(b)
            _fire_srcdst(jnp.minimum(sid + 2, NCHUNK - 1), b)

        _fire_srcdst(base, 0)
        _fire_srcdst(jnp.minimum(base + 1, NCHUNK - 1), 1)
        _wait_srcdst(0)
        _fire_gathers(0)

        def _pair(i, c):
            _step(2 * i, 0)
            _step(2 * i + 1, 1)
            return c
        lax.fori_loop(0, CPT // 2, _pair, 0)

        _wait_srcdst(1)
        _wait_gathers(CPT % 2)
        _wait_scatters(1 - CPT % 2)

    @pl.when(cid == 0)
    def _():
        _mainloop(hA_hbm, True)

    @pl.when(cid == 1)
    def _():
        _mainloop(hB_hbm, False)

    plsc.subcore_barrier()

    @pl.when(cid == 0)
    def _():
        pltpu.sync_copy(acc_sh.at[pl.ds(tid * ROWS_T, ROWS_T)],
                        acc_out.at[pl.ds(tid * ROWS_T, ROWS_T), pl.ds(0, HH)])

    @pl.when(cid == 1)
    def _():
        pltpu.sync_copy(acc_sh.at[pl.ds(tid * ROWS_T, ROWS_T)],
                        acc_out.at[pl.ds(tid * ROWS_T, ROWS_T), pl.ds(HH, HH)])

    @pl.when((cid == 0) & (tid < 2))
    def _():
        pltpu.sync_copy(den_sh.at[pl.ds(tid * DHALF, DHALF)],
                        den_out.at[pl.ds(tid * DHALF, DHALF)])


_sc_gat = pl.kernel(
    _sc_body,
    out_type=[
        jax.ShapeDtypeStruct((N, 128), jnp.float32),
        jax.ShapeDtypeStruct((N,), jnp.float32),
    ],
    mesh=plsc.VectorSubcoreMesh(core_axis_name="c", subcore_axis_name="s"),
    compiler_params=pltpu.CompilerParams(use_tc_tiling_on_sc=False),
    scratch_types=(
        [pltpu.VMEM_SHARED((N, HH), jnp.float32),
         pltpu.VMEM_SHARED((N,), jnp.float32)]
        + 2 * [
            pltpu.VMEM((KROW, IDXW), jnp.int32),
            pltpu.VMEM((KROW, IDXW), jnp.int32),
            pltpu.VMEM((CH,), jnp.float32),
            pltpu.VMEM((CH,), jnp.float32),
            pltpu.VMEM((CH, HH), jnp.float32),
            pltpu.VMEM((KROW, IDXW), jnp.int32),
            pltpu.VMEM((CH,), jnp.float32),
        ]
        + [
            pltpu.VMEM((1008,), jnp.float32),
            pltpu.VMEM((16,), jnp.float32),
            pltpu.SemaphoreType.DMA,
            pltpu.SemaphoreType.DMA,
            pltpu.SemaphoreType.DMA,
        ]
    ),
)


def _leaky_bound(m):
    z = m[0, 0] + m[0, 1]
    z = jnp.maximum(z, 0.2 * z)
    return jnp.broadcast_to(z, (16,))


@jax.jit
def kernel(x, edge_index, batch, W1, a1s, a1d, b1, W2, a2s, a2d, b2, Wl, bl):
    src = edge_index[0].reshape(NCHUNK, KROW, IDXW)
    dst = edge_index[1].reshape(NCHUNK, KROW, IDXW)

    A1 = jnp.stack([a1s, a1d], axis=1)
    A2 = jnp.stack([a2s, a2d], axis=1)

    h0, h1, asvo, advo, m = _k1(x, W1, A1)
    acc, den = _sc_gat(src, dst, asvo.reshape(N), advo.reshape(N),
                       h0, h1, _leaky_bound(m))
    den3 = den.reshape(NBLK, 1, RB)

    h0, h1, asvo, advo, m = _k2(acc, den3, b1.reshape(1, H), W2, A2)
    acc, den = _sc_gat(src, dst, asvo.reshape(N), advo.reshape(N),
                       h0, h1, _leaky_bound(m))
    den3 = den.reshape(NBLK, 1, RB)

    batch3 = batch.reshape(NBLK, 1, RB)
    return _k3(acc, den3, b2.reshape(1, H), batch3,
               Wl, bl.reshape(1, OUT))


# speedup vs baseline: 54.1623x; 1.0088x over previous
import functools

import jax
import jax.numpy as jnp
from jax import lax
from jax.experimental import pallas as pl
from jax.experimental.pallas import tpu as pltpu
from jax.experimental.pallas import tpu_sc as plsc

N = 50000
E = 800000
IN_DIM = 768
H = 64
HH = 32
G = 256
OUT = 2

RB = 2000
NBLK = N // RB
RB2 = 10000
NBLK2 = N // RB2

NSC = 2
NT = 16
IDXW = 128
KROW = 2
CH = IDXW * KROW
NCHUNK = E // CH
CPT = (NCHUNK + NT - 1) // NT
ROWS_T = N // NT
DHALF = N // 2

_EPS = 1e-16


def _tc_finish(h, A_ref, i, h0_ref, h1_ref, as_ref, ad_ref, m_ref):
    aux = jnp.dot(h, A_ref[...], preferred_element_type=jnp.float32)
    h0_ref[...] = h[:, :HH]
    h1_ref[...] = h[:, HH:]
    as_ref[...] = aux[:, 0:1].T[None]
    ad_ref[...] = aux[:, 1:2].T[None]
    mb = jnp.max(aux, axis=0)[None]

    @pl.when(i == 0)
    def _():
        m_ref[...] = mb

    @pl.when(i > 0)
    def _():
        m_ref[...] = jnp.maximum(m_ref[...], mb)


def _k1_body(x_ref, W_ref, A_ref, h0_ref, h1_ref, as_ref, ad_ref, m_ref):
    i = pl.program_id(0)
    h = jnp.dot(x_ref[...], W_ref[...].T, preferred_element_type=jnp.float32)
    _tc_finish(h, A_ref, i, h0_ref, h1_ref, as_ref, ad_ref, m_ref)


def _norm_in(accd_ref, den_ref, b_ref):
    inv = jnp.reshape(1.0 / (den_ref[0] + _EPS), (-1, 1))
    hin = accd_ref[...][:, :H]
    return jnp.maximum(hin * inv + b_ref[...], 0.0)


def _k2_body(accd_ref, den_ref, b_ref, W_ref, A_ref,
             h0_ref, h1_ref, as_ref, ad_ref, m_ref):
    i = pl.program_id(0)
    hin = _norm_in(accd_ref, den_ref, b_ref)
    h = jnp.dot(hin, W_ref[...].T, preferred_element_type=jnp.float32)
    _tc_finish(h, A_ref, i, h0_ref, h1_ref, as_ref, ad_ref, m_ref)


def _k3_body(accd_ref, den_ref, b_ref, batch_ref, Wl_ref, bl_ref,
             out_ref, sums_ref):
    i = pl.program_id(0)
    hin = _norm_in(accd_ref, den_ref, b_ref)
    hin = jnp.concatenate([hin, jnp.ones((RB2, 1), jnp.float32)], axis=1)
    bb = jnp.reshape(batch_ref[0], (RB2, 1))
    gid = lax.broadcasted_iota(jnp.int32, (1, G), 1)
    onehot = (bb == gid).astype(jnp.float32)
    part = lax.dot_general(onehot, hin, (((0,), (0,)), ((), ())),
                           preferred_element_type=jnp.float32)

    @pl.when(i == 0)
    def _():
        sums_ref[...] = part

    @pl.when(i > 0)
    def _():
        sums_ref[...] = sums_ref[...] + part

    @pl.when(i == NBLK2 - 1)
    def _():
        s = sums_ref[...]
        g = s[:, :H] / jnp.maximum(s[:, H:], 1.0)
        out_ref[...] = (
            jnp.dot(g, Wl_ref[...].T, preferred_element_type=jnp.float32)
            + bl_ref[...])


_HEAD_OUT_SPECS = [
    pl.BlockSpec((RB, HH), lambda i: (i, 0)),
    pl.BlockSpec((RB, HH), lambda i: (i, 0)),
    pl.BlockSpec((1, 1, RB), lambda i: (i, 0, 0)),
    pl.BlockSpec((1, 1, RB), lambda i: (i, 0, 0)),
    pl.BlockSpec((1, 2), lambda i: (0, 0)),
]
_HEAD_OUT_SHAPE = [
    jax.ShapeDtypeStruct((N, HH), jnp.float32),
    jax.ShapeDtypeStruct((N, HH), jnp.float32),
    jax.ShapeDtypeStruct((NBLK, 1, RB), jnp.float32),
    jax.ShapeDtypeStruct((NBLK, 1, RB), jnp.float32),
    jax.ShapeDtypeStruct((1, 2), jnp.float32),
]

_k1 = pl.pallas_call(
    _k1_body,
    grid=(NBLK,),
    in_specs=[
        pl.BlockSpec((RB, IN_DIM), lambda i: (i, 0)),
        pl.BlockSpec((H, IN_DIM), lambda i: (0, 0)),
        pl.BlockSpec((H, 2), lambda i: (0, 0)),
    ],
    out_specs=_HEAD_OUT_SPECS,
    out_shape=_HEAD_OUT_SHAPE,
)

_HEAD_OUT_SPECS2 = [
    pl.BlockSpec((RB2, HH), lambda i: (i, 0)),
    pl.BlockSpec((RB2, HH), lambda i: (i, 0)),
    pl.BlockSpec((1, 1, RB2), lambda i: (i, 0, 0)),
    pl.BlockSpec((1, 1, RB2), lambda i: (i, 0, 0)),
    pl.BlockSpec((1, 2), lambda i: (0, 0)),
]
_HEAD_OUT_SHAPE2 = [
    jax.ShapeDtypeStruct((N, HH), jnp.float32),
    jax.ShapeDtypeStruct((N, HH), jnp.float32),
    jax.ShapeDtypeStruct((NBLK2, 1, RB2), jnp.float32),
    jax.ShapeDtypeStruct((NBLK2, 1, RB2), jnp.float32),
    jax.ShapeDtypeStruct((1, 2), jnp.float32),
]

_k2 = pl.pallas_call(
    _k2_body,
    grid=(NBLK2,),
    in_specs=[
        pl.BlockSpec((RB2, 128), lambda i: (i, 0)),
        pl.BlockSpec((1, 1, RB2), lambda i: (i, 0, 0)),
        pl.BlockSpec((1, H), lambda i: (0, 0)),
        pl.BlockSpec((H, H), lambda i: (0, 0)),
        pl.BlockSpec((H, 2), lambda i: (0, 0)),
    ],
    out_specs=_HEAD_OUT_SPECS2,
    out_shape=_HEAD_OUT_SHAPE2,
)

_k3 = pl.pallas_call(
    _k3_body,
    grid=(NBLK2,),
    in_specs=[
        pl.BlockSpec((RB2, 128), lambda i: (i, 0)),
        pl.BlockSpec((1, 1, RB2), lambda i: (i, 0, 0)),
        pl.BlockSpec((1, H), lambda i: (0, 0)),
        pl.BlockSpec((1, 1, RB2), lambda i: (i, 0, 0)),
        pl.BlockSpec((OUT, H), lambda i: (0, 0)),
        pl.BlockSpec((1, OUT), lambda i: (0, 0)),
    ],
    out_specs=pl.BlockSpec((G, OUT), lambda i: (0, 0)),
    out_shape=jax.ShapeDtypeStruct((G, OUT), jnp.float32),
    scratch_shapes=[pltpu.VMEM((G, H + 1), jnp.float32)],
)


def _sc_body(src_hbm, dst_hbm, asv_hbm, adv_hbm, hA_hbm, hB_hbm, m_hbm,
             acc_out, den_out,
             acc_sh, den_sh,
             srcv0, dstv0, asg0, adg0, hrow0, dstc0, eev0,
             srcv1, dstv1, asg1, adg1, hrow1, dstc1, eev1,
             zden, m_v, sem_i, sem_g, sem_s):
    cid = lax.axis_index("c")
    tid = lax.axis_index("s")
    bufs = ((srcv0, dstv0, asg0, adg0, hrow0, dstc0, eev0),
            (srcv1, dstv1, asg1, adg1, hrow1, dstc1, eev1))

    def _zb(r, c):
        hrow0[r, pl.ds(0, 16)] = jnp.zeros((16,), jnp.float32)
        hrow0[r, pl.ds(16, 16)] = jnp.zeros((16,), jnp.float32)
        return c
    lax.fori_loop(0, CH, _zb, 0)

    def _zd(r, c):
        zden[pl.ds(r * 16, 16)] = jnp.zeros((16,), jnp.float32)
        return c
    lax.fori_loop(0, 1008 // 16, _zd, 0)

    for k in range(ROWS_T // CH):
        pltpu.sync_copy(hrow0, acc_sh.at[pl.ds(tid * ROWS_T + k * CH, CH)])
    _rem = ROWS_T - (ROWS_T // CH) * CH
    pltpu.sync_copy(hrow0.at[pl.ds(0, _rem)],
                    acc_sh.at[pl.ds(tid * ROWS_T + ROWS_T - _rem, _rem)])

    @pl.when(tid < 2)
    def _():
        def _zdn(k, c):
            pltpu.sync_copy(zden.at[pl.ds(0, 1000)],
                            den_sh.at[pl.ds(tid * DHALF + k * 1000, 1000)])
            return c
        lax.fori_loop(0, DHALF // 1000, _zdn, 0)

    pltpu.sync_copy(m_hbm, m_v)
    plsc.subcore_barrier()

    m_s = m_v[pl.ds(0, 16)][0]

    def _fire_srcdst(sid, b):
        pltpu.async_copy(src_hbm.at[sid], bufs[b][0], sem_i)
        pltpu.async_copy(dst_hbm.at[sid], bufs[b][1], sem_i)

    def _wait_srcdst(b):
        pltpu.make_async_copy(src_hbm.at[0], bufs[b][0], sem_i).wait()
        pltpu.make_async_copy(dst_hbm.at[0], bufs[b][1], sem_i).wait()

    def _mainloop(htab, do_den):
        base = tid * CPT

        def _fire_gathers(b):
            srcv, dstv, asg, adg, hrow = bufs[b][:5]
            for j in range(KROW):
                pltpu.async_copy(asv_hbm.at[srcv.at[j]],
                                 asg.at[pl.ds(j * IDXW, IDXW)], sem_g)
                pltpu.async_copy(adv_hbm.at[dstv.at[j]],
                                 adg.at[pl.ds(j * IDXW, IDXW)], sem_g)
                pltpu.async_copy(htab.at[srcv.at[j]],
                                 hrow.at[pl.ds(j * IDXW, IDXW)], sem_g)

        def _wait_gathers(b):
            srcv, dstv, asg, adg, hrow = bufs[b][:5]
            for j in range(KROW):
                pltpu.make_async_copy(asv_hbm.at[srcv.at[j]],
                                      asg.at[pl.ds(j * IDXW, IDXW)],
                                      sem_g).wait()
                pltpu.make_async_copy(adv_hbm.at[dstv.at[j]],
                                      adg.at[pl.ds(j * IDXW, IDXW)],
                                      sem_g).wait()
                pltpu.make_async_copy(htab.at[srcv.at[j]],
                                      hrow.at[pl.ds(j * IDXW, IDXW)],
                                      sem_g).wait()

        def _fire_scatters(b):
            hrow, dstc, eev = bufs[b][4:7]
            for j in range(KROW):
                pltpu.async_copy(hrow.at[pl.ds(j * IDXW, IDXW)],
                                 acc_sh.at[dstc.at[j]], sem_s, add=True)
            if do_den:
                for j in range(KROW):
                    pltpu.async_copy(eev.at[pl.ds(j * IDXW, IDXW)],
                                     den_sh.at[dstc.at[j]], sem_s, add=True)

        def _wait_scatters(b):
            hrow, dstc, eev = bufs[b][4:7]
            for j in range(KROW):
                pltpu.make_async_copy(hrow.at[pl.ds(j * IDXW, IDXW)],
                                      acc_sh.at[dstc.at[j]], sem_s).wait()
            if do_den:
                for j in range(KROW):
                    pltpu.make_async_copy(eev.at[pl.ds(j * IDXW, IDXW)],
                                          den_sh.at[dstc.at[j]], sem_s).wait()

        def _step(k, b):
            sid = base + k
            _wait_srcdst(1 - b)

            @pl.when(k > 0)
            def _():
                _wait_scatters(1 - b)

            _fire_gathers(1 - b)
            _wait_gathers(b)

            srcv, dstv, asg, adg, hrow, dstc, eev = bufs[b]
            gate = jnp.where(sid < NCHUNK,
                             jnp.full((16,), 1.0, jnp.float32),
                             jnp.full((16,), 0.0, jnp.float32))

            for v in range(CH // 16):
                a = asg[pl.ds(v * 16, 16)]
                bb = adg[pl.ds(v * 16, 16)]
                z = a + bb
                z = jnp.maximum(z, 0.2 * z)
                eev[pl.ds(v * 16, 16)] = jnp.exp(z - m_s) * gate

            def _scale(q, c2):
                ev = eev[pl.ds(q * 16, 16)]
                for l in range(16):
                    r = q * 16 + l
                    es = ev[l]
                    hrow[r, pl.ds(0, 16)] = hrow[r, pl.ds(0, 16)] * es
                    hrow[r, pl.ds(16, 16)] = hrow[r, pl.ds(16, 16)] * es
                return c2
            lax.fori_loop(0, CH // 16, _scale, 0)

            for j in range(KROW):
                for v in range(IDXW // 16):
                    dstc[j, pl.ds(v * 16, 16)] = dstv[j, pl.ds(v * 16, 16)]

            _fire_scatters(b)
            _fire_srcdst(jnp.minimum(sid + 2, NCHUNK - 1), b)

        _fire_srcdst(base, 0)
        _fire_srcdst(jnp.minimum(base + 1, NCHUNK - 1), 1)
        _wait_srcdst(0)
        _fire_gathers(0)

        def _pair(i, c):
            _step(2 * i, 0)
            _step(2 * i + 1, 1)
            return c
        lax.fori_loop(0, CPT // 2, _pair, 0)

        _wait_srcdst(1)
        _wait_gathers(CPT % 2)
        _wait_scatters(1 - CPT % 2)

    @pl.when(cid == 0)
    def _():
        _mainloop(hA_hbm, True)

    @pl.when(cid == 1)
    def _():
        _mainloop(hB_hbm, False)

    plsc.subcore_barrier()

    @pl.when(cid == 0)
    def _():
        pltpu.sync_copy(acc_sh.at[pl.ds(tid * ROWS_T, ROWS_T)],
                        acc_out.at[pl.ds(tid * ROWS_T, ROWS_T), pl.ds(0, HH)])

    @pl.when(cid == 1)
    def _():
        pltpu.sync_copy(acc_sh.at[pl.ds(tid * ROWS_T, ROWS_T)],
                        acc_out.at[pl.ds(tid * ROWS_T, ROWS_T), pl.ds(HH, HH)])

    @pl.when((cid == 0) & (tid < 2))
    def _():
        pltpu.sync_copy(den_sh.at[pl.ds(tid * DHALF, DHALF)],
                        den_out.at[pl.ds(tid * DHALF, DHALF)])


_sc_gat = pl.kernel(
    _sc_body,
    out_type=[
        jax.ShapeDtypeStruct((N, 128), jnp.float32),
        jax.ShapeDtypeStruct((N,), jnp.float32),
    ],
    mesh=plsc.VectorSubcoreMesh(core_axis_name="c", subcore_axis_name="s"),
    compiler_params=pltpu.CompilerParams(use_tc_tiling_on_sc=False),
    scratch_types=(
        [pltpu.VMEM_SHARED((N, HH), jnp.float32),
         pltpu.VMEM_SHARED((N,), jnp.float32)]
        + 2 * [
            pltpu.VMEM((KROW, IDXW), jnp.int32),
            pltpu.VMEM((KROW, IDXW), jnp.int32),
            pltpu.VMEM((CH,), jnp.float32),
            pltpu.VMEM((CH,), jnp.float32),
            pltpu.VMEM((CH, HH), jnp.float32),
            pltpu.VMEM((KROW, IDXW), jnp.int32),
            pltpu.VMEM((CH,), jnp.float32),
        ]
        + [
            pltpu.VMEM((1008,), jnp.float32),
            pltpu.VMEM((16,), jnp.float32),
            pltpu.SemaphoreType.DMA,
            pltpu.SemaphoreType.DMA,
            pltpu.SemaphoreType.DMA,
        ]
    ),
)


def _leaky_bound(m):
    z = m[0, 0] + m[0, 1]
    z = jnp.maximum(z, 0.2 * z)
    return jnp.broadcast_to(z, (16,))


@jax.jit
def kernel(x, edge_index, batch, W1, a1s, a1d, b1, W2, a2s, a2d, b2, Wl, bl):
    src = edge_index[0].reshape(NCHUNK, KROW, IDXW)
    dst = edge_index[1].reshape(NCHUNK, KROW, IDXW)

    A1 = jnp.stack([a1s, a1d], axis=1)
    A2 = jnp.stack([a2s, a2d], axis=1)

    h0, h1, asvo, advo, m = _k1(x, W1, A1)
    acc, den = _sc_gat(src, dst, asvo.reshape(N), advo.reshape(N),
                       h0, h1, _leaky_bound(m))
    den3 = den.reshape(NBLK2, 1, RB2)

    h0, h1, asvo, advo, m = _k2(acc, den3, b1.reshape(1, H), W2, A2)
    acc, den = _sc_gat(src, dst, asvo.reshape(N), advo.reshape(N),
                       h0, h1, _leaky_bound(m))
    den3 = den.reshape(NBLK2, 1, RB2)

    batch3 = batch.reshape(NBLK2, 1, RB2)
    return _k3(acc, den3, b2.reshape(1, H), batch3,
               Wl, bl.reshape(1, OUT))
